# baseline (speedup 1.0000x reference)
import jax
import jax.numpy as jnp
from jax import lax
from jax.experimental import pallas as pl
from jax.experimental.pallas import tpu as pltpu
from jax.experimental.pallas import tpu_sc as plsc

N = 10000
E = 320000
H = 8
HD = 16
AUG = 144
ALW = 16
ALPHA = 0.2

NC = 2
NS = 16
NW = NC * NS
EPW = E // NW
C = 80
NCHUNK = EPW // C
RPT = N // NS
RQ = 25
NQ = RPT // RQ


def _tc_prep(x_ref, wt_ref, a1p_ref, a2p_ref, haug_ref, alph_ref):
    h = jnp.dot(x_ref[...], wt_ref[...], preferred_element_type=jnp.float32)
    al2 = jnp.dot(h, a2p_ref[...], preferred_element_type=jnp.float32)
    haug_ref[...] = jnp.concatenate([h, al2], axis=1)
    alph_ref[...] = jnp.dot(h, a1p_ref[...], preferred_element_type=jnp.float32)


def _tc_combine(pa_ref, pb_ref, haug_ref, exp_ref, out_ref):
    a = pa_ref[...]
    b = pb_ref[...]
    num = a[:, :128] + b[:, :128]
    den8 = a[:, 128:136] + b[:, 128:136]
    deg = a[:, 136:137] + b[:, 136:137]
    denf = jnp.dot(den8 + (jnp.float32(N) - deg), exp_ref[...],
                   preferred_element_type=jnp.float32)
    h = haug_ref[...][:, :128]
    out_ref[...] = jnp.where(deg > 0, num / denf, h)


def _sc_edges(haug_hbm, alph_hbm, ei_hbm, ea_hbm, out_hbm,
              acc, sdbuf0, sdbuf1, sdbuf2, eabuf, a1buf, hdbuf, paybuf,
              semsd0, semsd1, semsd2, semea0, semea1,
              semhd0, semhd1, sema10, sema11, sempay):
    c = lax.axis_index("c")
    s = lax.axis_index("s")
    wid = s * NC + c
    sdbuf = (sdbuf0, sdbuf1, sdbuf2)
    semsd = (semsd0, semsd1, semsd2)
    semea = (semea0, semea1)
    semhd = (semhd0, semhd1)
    sema1 = (sema10, sema11)

    z16 = jnp.zeros((16,), jnp.float32)

    def zero_paybuf(i, carry):
        for j in range(0, AUG, 16):
            paybuf[i, pl.ds(j, 16)] = z16
        return carry

    lax.fori_loop(0, C, zero_paybuf, 0)

    def zero_acc(q, carry):
        pltpu.make_async_copy(paybuf.at[pl.ds(0, RQ)],
                              acc.at[pl.ds(s * RPT + q * RQ, RQ)],
                              semsd0).start()
        return carry

    def zero_drain(q, carry):
        pltpu.make_async_copy(paybuf.at[pl.ds(0, RQ)],
                              acc.at[pl.ds(s * RPT + q * RQ, RQ)],
                              semsd0).wait()
        return carry

    lax.fori_loop(0, NQ, zero_acc, 0)
    lax.fori_loop(0, NQ, zero_drain, 0)
    plsc.subcore_barrier()

    ebase = wid * EPW
    iota16 = lax.iota(jnp.int32, 16)

    def lin_sd(ci, s3):
        cb = ebase + ci * C
        return pltpu.make_async_copy(ei_hbm.at[:, pl.ds(cb, C)],
                                     sdbuf[s3], semsd[s3])

    def lin_ea(ci, s2):
        cb = ebase + ci * C
        return pltpu.make_async_copy(ea_hbm.at[pl.ds(cb, C)],
                                     eabuf.at[pl.ds(s2 * C, C)], semea[s2])

    def gather_hd(s3, s2):
        return pltpu.make_async_copy(haug_hbm.at[sdbuf[s3].at[1]],
                                     hdbuf.at[pl.ds(s2 * C, C)], semhd[s2])

    def gather_a1(s3, s2):
        return pltpu.make_async_copy(alph_hbm.at[sdbuf[s3].at[0]],
                                     a1buf.at[pl.ds(s2 * C, C)], sema1[s2])

    def scat(s3):
        return pltpu.make_async_copy(paybuf, acc.at[sdbuf[s3].at[0]], sempay)

    def col(k):
        return jnp.full((16,), k, jnp.int32)

    def compute(off):
        def grp(g, carry):
            pidx = g * 16 + iota16
            eidx = pidx + off

            ea_k = [plsc.load_gather(eabuf, [eidx, col(k)])
                    for k in range(HD)]

            @plsc.parallel_loop(0, H, unroll=1)
            def head(hh):
                base = hh * 16
                a1v = plsc.load_gather(a1buf, [eidx, col(0) + hh])
                a2v = plsc.load_gather(hdbuf, [eidx, col(128) + hh])
                sv = a1v + a2v
                sv = jnp.where(sv >= 0, sv, ALPHA * sv)
                hvals = [plsc.load_gather(hdbuf, [eidx, col(k) + base])
                         for k in range(HD)]
                prods = [ea_k[k] * hvals[k] for k in range(HD)]
                while len(prods) > 1:
                    prods = [prods[i] + prods[i + 1]
                             for i in range(0, len(prods), 2)]
                pv = jnp.exp(sv + prods[0])
                for k in range(HD):
                    plsc.store_scatter(paybuf, [pidx, col(k) + base],
                                       pv * hvals[k])
                plsc.store_scatter(paybuf, [pidx, col(128) + hh], pv)

            plsc.store_scatter(paybuf, [pidx, col(136)],
                               jnp.ones((16,), jnp.float32))
            return carry

        lax.fori_loop(0, C // 16, grp, 0)

    la = lin_sd(0, 0)
    lb = lin_ea(0, 0)
    la.start()
    lb.start()
    la.wait()
    lb.wait()
    gather_hd(0, 0).start()
    gather_a1(0, 0).start()
    lin_sd(1, 1).start()
    lin_ea(1, 1).start()

    def pipe(ci, carry):
        b2 = lax.rem(ci, 2)
        b3 = lax.rem(ci, 3)
        nxt = ci + 1
        n2 = lax.rem(nxt, 2)
        n3 = lax.rem(nxt, 3)
        p3 = lax.rem(ci + 2, 3)
        f2 = b2

        for v3 in range(3):
            for v2 in range(2):
                @pl.when(jnp.logical_and(
                        nxt < NCHUNK,
                        jnp.logical_and(n3 == v3, n2 == v2)))
                def _(v3=v3, v2=v2):
                    lin_sd(nxt, v3).wait()
                    lin_ea(nxt, v2).wait()
                    gather_hd(v3, v2).start()
                    gather_a1(v3, v2).start()

        for v3 in range(3):
            for v2 in range(2):
                @pl.when(jnp.logical_and(b3 == v3, b2 == v2))
                def _(v3=v3, v2=v2):
                    gather_hd(v3, v2).wait()
                    gather_a1(v3, v2).wait()

        for v3 in range(3):
            @pl.when(jnp.logical_and(ci > 0, p3 == v3))
            def _(v3=v3):
                scat(v3).wait()

        compute(b2 * C)

        for v3 in range(3):
            @pl.when(b3 == v3)
            def _(v3=v3):
                scat(v3).start(add=True)

        for v3 in range(3):
            @pl.when(jnp.logical_and(ci + 2 < NCHUNK, p3 == v3))
            def _(v3=v3):
                lin_sd(ci + 2, v3).start()

        for v2 in range(2):
            @pl.when(jnp.logical_and(ci + 2 < NCHUNK, f2 == v2))
            def _(v2=v2):
                lin_ea(ci + 2, v2).start()

        return carry

    lax.fori_loop(0, NCHUNK, pipe, 0)
    scat((NCHUNK - 1) % 3).wait()

    plsc.subcore_barrier()

    def writeout(q, carry):
        rs = s * RPT + q * RQ
        pltpu.make_async_copy(acc.at[pl.ds(rs, RQ)],
                              out_hbm.at[c, pl.ds(rs, RQ)], semsd0).start()
        return carry

    def writeout_drain(q, carry):
        rs = s * RPT + q * RQ
        pltpu.make_async_copy(acc.at[pl.ds(rs, RQ)],
                              out_hbm.at[c, pl.ds(rs, RQ)], semsd0).wait()
        return carry

    lax.fori_loop(0, NQ, writeout, 0)
    lax.fori_loop(0, NQ, writeout_drain, 0)


def kernel(node_features, edge_index, edge_attr, W, a):
    x = node_features
    a1 = a[:HD, 0]
    a2 = a[HD:, 0]
    eye8 = jnp.eye(H, dtype=jnp.float32)
    A1p = jnp.concatenate(
        [jnp.kron(eye8, a1[:, None]), jnp.zeros((128, ALW - H), jnp.float32)],
        axis=1)
    A2p = jnp.concatenate(
        [jnp.kron(eye8, a2[:, None]),
         jnp.zeros((128, AUG - 128 - H), jnp.float32)], axis=1)
    expand = jnp.kron(eye8, jnp.ones((1, HD), jnp.float32))

    haug, alph = pl.pallas_call(
        _tc_prep,
        grid=(10,),
        in_specs=[
            pl.BlockSpec((1000, 128), lambda i: (i, 0)),
            pl.BlockSpec((128, 128), lambda i: (0, 0)),
            pl.BlockSpec((128, ALW), lambda i: (0, 0)),
            pl.BlockSpec((128, AUG - 128), lambda i: (0, 0)),
        ],
        out_specs=[
            pl.BlockSpec((1000, AUG), lambda i: (i, 0)),
            pl.BlockSpec((1000, ALW), lambda i: (i, 0)),
        ],
        out_shape=[
            jax.ShapeDtypeStruct((N, AUG), jnp.float32),
            jax.ShapeDtypeStruct((N, ALW), jnp.float32),
        ],
    )(x, W.T, A1p, A2p)

    mesh = plsc.VectorSubcoreMesh(core_axis_name="c", subcore_axis_name="s")
    sc_fn = pl.kernel(
        _sc_edges,
        mesh=mesh,
        compiler_params=pltpu.CompilerParams(
            needs_layout_passes=False, use_tc_tiling_on_sc=False),
        out_type=jax.ShapeDtypeStruct((NC, N, AUG), jnp.float32),
        scratch_types=[
            pltpu.VMEM_SHARED((N, AUG), jnp.float32),
            pltpu.VMEM((2, C), jnp.int32),
            pltpu.VMEM((2, C), jnp.int32),
            pltpu.VMEM((2, C), jnp.int32),
            pltpu.VMEM((2 * C, HD), jnp.float32),
            pltpu.VMEM((2 * C, ALW), jnp.float32),
            pltpu.VMEM((2 * C, AUG), jnp.float32),
            pltpu.VMEM((C, AUG), jnp.float32),
        ] + [pltpu.SemaphoreType.DMA] * 10,
    )
    partials = sc_fn(haug, alph, edge_index, edge_attr)

    out = pl.pallas_call(
        _tc_combine,
        grid=(10,),
        in_specs=[
            pl.BlockSpec((1000, AUG), lambda i: (i, 0)),
            pl.BlockSpec((1000, AUG), lambda i: (i, 0)),
            pl.BlockSpec((1000, AUG), lambda i: (i, 0)),
            pl.BlockSpec((8, 128), lambda i: (0, 0)),
        ],
        out_specs=pl.BlockSpec((1000, 128), lambda i: (i, 0)),
        out_shape=jax.ShapeDtypeStruct((N, 128), jnp.float32),
    )(partials[0], partials[1], haug, expand)
    return out


# speedup vs baseline: 1.0908x; 1.0908x over previous
import jax
import jax.numpy as jnp
from jax import lax
from jax.experimental import pallas as pl
from jax.experimental.pallas import tpu as pltpu
from jax.experimental.pallas import tpu_sc as plsc

N = 10000
E = 320000
H = 8
HD = 16
AUG = 144
ALW = 16
ALPHA = 0.2

NC = 2
NS = 16
NW = NC * NS
EPW = E // NW
C = 80
NCHUNK = EPW // C
RPT = N // NS
RQ = 25
NQ = RPT // RQ


def _tc_prep(x_ref, wt_ref, a1p_ref, a2p_ref, haug_ref, alph_ref):
    h = jnp.dot(x_ref[...], wt_ref[...], preferred_element_type=jnp.float32)
    al2 = jnp.dot(h, a2p_ref[...], preferred_element_type=jnp.float32)
    haug_ref[...] = jnp.concatenate([h, al2], axis=1)
    alph_ref[...] = jnp.dot(h, a1p_ref[...], preferred_element_type=jnp.float32)


def _tc_combine(pa_ref, pb_ref, haug_ref, exp_ref, out_ref):
    a = pa_ref[...]
    b = pb_ref[...]
    num = a[:, :128] + b[:, :128]
    den8 = a[:, 128:136] + b[:, 128:136]
    deg = a[:, 136:137] + b[:, 136:137]
    denf = jnp.dot(den8 + (jnp.float32(N) - deg), exp_ref[...],
                   preferred_element_type=jnp.float32)
    h = haug_ref[...][:, :128]
    out_ref[...] = jnp.where(deg > 0, num / denf, h)


def _sc_edges(haug_hbm, alph_hbm, ei_hbm, ea_hbm, out_hbm,
              acc, sdbuf0, sdbuf1, sdbuf2, eabuf, a1buf, hdbuf, paybuf,
              semsd0, semsd1, semsd2, semea0, semea1,
              semhd0, semhd1, sema10, sema11, sempay):
    c = lax.axis_index("c")
    s = lax.axis_index("s")
    wid = s * NC + c
    sdbuf = (sdbuf0, sdbuf1, sdbuf2)
    semsd = (semsd0, semsd1, semsd2)
    semea = (semea0, semea1)
    semhd = (semhd0, semhd1)
    sema1 = (sema10, sema11)

    z16 = jnp.zeros((16,), jnp.float32)

    def zero_paybuf(i, carry):
        for j in range(0, AUG, 16):
            paybuf[i, pl.ds(j, 16)] = z16
        return carry

    lax.fori_loop(0, C, zero_paybuf, 0)

    def zero_acc(q, carry):
        pltpu.make_async_copy(paybuf.at[pl.ds(0, RQ)],
                              acc.at[pl.ds(s * RPT + q * RQ, RQ)],
                              semsd0).start()
        return carry

    def zero_drain(q, carry):
        pltpu.make_async_copy(paybuf.at[pl.ds(0, RQ)],
                              acc.at[pl.ds(s * RPT + q * RQ, RQ)],
                              semsd0).wait()
        return carry

    lax.fori_loop(0, NQ, zero_acc, 0)
    lax.fori_loop(0, NQ, zero_drain, 0)
    plsc.subcore_barrier()

    ebase = wid * EPW
    iota16 = lax.iota(jnp.int32, 16)

    def lin_sd(ci, s3):
        cb = ebase + ci * C
        return pltpu.make_async_copy(ei_hbm.at[:, pl.ds(cb, C)],
                                     sdbuf[s3], semsd[s3])

    def lin_ea(ci, s2):
        cb = ebase + ci * C
        return pltpu.make_async_copy(ea_hbm.at[pl.ds(cb, C)],
                                     eabuf.at[pl.ds(s2 * C, C)], semea[s2])

    def gather_hd(s3, s2):
        return pltpu.make_async_copy(haug_hbm.at[sdbuf[s3].at[1]],
                                     hdbuf.at[pl.ds(s2 * C, C)], semhd[s2])

    def gather_a1(s3, s2):
        return pltpu.make_async_copy(alph_hbm.at[sdbuf[s3].at[0]],
                                     a1buf.at[pl.ds(s2 * C, C)], sema1[s2])

    def scat(s3):
        return pltpu.make_async_copy(paybuf, acc.at[sdbuf[s3].at[0]], sempay)

    def col(k):
        return jnp.full((16,), k, jnp.int32)

    def compute(off):
        def grp(g, carry):
            pidx = g * 16 + iota16
            eidx = pidx + off

            ea_k = [plsc.load_gather(eabuf, [eidx, col(k)])
                    for k in range(HD)]

            def head(hh, carry2):
                base = hh * 16
                a1v = plsc.load_gather(a1buf, [eidx, col(0) + hh])
                a2v = plsc.load_gather(hdbuf, [eidx, col(128) + hh])
                sv = a1v + a2v
                sv = jnp.where(sv >= 0, sv, ALPHA * sv)
                hvals = [plsc.load_gather(hdbuf, [eidx, col(k) + base])
                         for k in range(HD)]
                prods = [ea_k[k] * hvals[k] for k in range(HD)]
                while len(prods) > 1:
                    prods = [prods[i] + prods[i + 1]
                             for i in range(0, len(prods), 2)]
                pv = jnp.exp(sv + prods[0])
                for k in range(HD):
                    plsc.store_scatter(paybuf, [pidx, col(k) + base],
                                       pv * hvals[k])
                plsc.store_scatter(paybuf, [pidx, col(128) + hh], pv)
                return carry2

            lax.fori_loop(0, H, head, 0)
            plsc.store_scatter(paybuf, [pidx, col(136)],
                               jnp.ones((16,), jnp.float32))
            return carry

        lax.fori_loop(0, C // 16, grp, 0)

    la = lin_sd(0, 0)
    lb = lin_ea(0, 0)
    la.start()
    lb.start()
    la.wait()
    lb.wait()
    gather_hd(0, 0).start()
    gather_a1(0, 0).start()
    lin_sd(1, 1).start()
    lin_ea(1, 1).start()

    def pipe(ci, carry):
        b2 = lax.rem(ci, 2)
        b3 = lax.rem(ci, 3)
        nxt = ci + 1
        n2 = lax.rem(nxt, 2)
        n3 = lax.rem(nxt, 3)
        p3 = lax.rem(ci + 2, 3)
        f2 = b2

        for v3 in range(3):
            for v2 in range(2):
                @pl.when(jnp.logical_and(
                        nxt < NCHUNK,
                        jnp.logical_and(n3 == v3, n2 == v2)))
                def _(v3=v3, v2=v2):
                    lin_sd(nxt, v3).wait()
                    lin_ea(nxt, v2).wait()
                    gather_hd(v3, v2).start()
                    gather_a1(v3, v2).start()

        for v3 in range(3):
            for v2 in range(2):
                @pl.when(jnp.logical_and(b3 == v3, b2 == v2))
                def _(v3=v3, v2=v2):
                    gather_hd(v3, v2).wait()
                    gather_a1(v3, v2).wait()

        for v3 in range(3):
            @pl.when(jnp.logical_and(ci > 0, p3 == v3))
            def _(v3=v3):
                scat(v3).wait()

        compute(b2 * C)

        for v3 in range(3):
            @pl.when(b3 == v3)
            def _(v3=v3):
                scat(v3).start(add=True)

        for v3 in range(3):
            @pl.when(jnp.logical_and(ci + 2 < NCHUNK, p3 == v3))
            def _(v3=v3):
                lin_sd(ci + 2, v3).start()

        for v2 in range(2):
            @pl.when(jnp.logical_and(ci + 2 < NCHUNK, f2 == v2))
            def _(v2=v2):
                lin_ea(ci + 2, v2).start()

        return carry

    lax.fori_loop(0, NCHUNK, pipe, 0)
    scat((NCHUNK - 1) % 3).wait()

    plsc.subcore_barrier()

    def writeout(q, carry):
        rs = s * RPT + q * RQ
        pltpu.make_async_copy(acc.at[pl.ds(rs, RQ)],
                              out_hbm.at[c, pl.ds(rs, RQ)], semsd0).start()
        return carry

    def writeout_drain(q, carry):
        rs = s * RPT + q * RQ
        pltpu.make_async_copy(acc.at[pl.ds(rs, RQ)],
                              out_hbm.at[c, pl.ds(rs, RQ)], semsd0).wait()
        return carry

    lax.fori_loop(0, NQ, writeout, 0)
    lax.fori_loop(0, NQ, writeout_drain, 0)


def kernel(node_features, edge_index, edge_attr, W, a):
    x = node_features
    a1 = a[:HD, 0]
    a2 = a[HD:, 0]
    eye8 = jnp.eye(H, dtype=jnp.float32)
    A1p = jnp.concatenate(
        [jnp.kron(eye8, a1[:, None]), jnp.zeros((128, ALW - H), jnp.float32)],
        axis=1)
    A2p = jnp.concatenate(
        [jnp.kron(eye8, a2[:, None]),
         jnp.zeros((128, AUG - 128 - H), jnp.float32)], axis=1)
    expand = jnp.kron(eye8, jnp.ones((1, HD), jnp.float32))

    haug, alph = pl.pallas_call(
        _tc_prep,
        grid=(10,),
        in_specs=[
            pl.BlockSpec((1000, 128), lambda i: (i, 0)),
            pl.BlockSpec((128, 128), lambda i: (0, 0)),
            pl.BlockSpec((128, ALW), lambda i: (0, 0)),
            pl.BlockSpec((128, AUG - 128), lambda i: (0, 0)),
        ],
        out_specs=[
            pl.BlockSpec((1000, AUG), lambda i: (i, 0)),
            pl.BlockSpec((1000, ALW), lambda i: (i, 0)),
        ],
        out_shape=[
            jax.ShapeDtypeStruct((N, AUG), jnp.float32),
            jax.ShapeDtypeStruct((N, ALW), jnp.float32),
        ],
    )(x, W.T, A1p, A2p)

    mesh = plsc.VectorSubcoreMesh(core_axis_name="c", subcore_axis_name="s")
    sc_fn = pl.kernel(
        _sc_edges,
        mesh=mesh,
        compiler_params=pltpu.CompilerParams(
            needs_layout_passes=False, use_tc_tiling_on_sc=False),
        out_type=jax.ShapeDtypeStruct((NC, N, AUG), jnp.float32),
        scratch_types=[
            pltpu.VMEM_SHARED((N, AUG), jnp.float32),
            pltpu.VMEM((2, C), jnp.int32),
            pltpu.VMEM((2, C), jnp.int32),
            pltpu.VMEM((2, C), jnp.int32),
            pltpu.VMEM((2 * C, HD), jnp.float32),
            pltpu.VMEM((2 * C, ALW), jnp.float32),
            pltpu.VMEM((2 * C, AUG), jnp.float32),
            pltpu.VMEM((C, AUG), jnp.float32),
        ] + [pltpu.SemaphoreType.DMA] * 10,
    )
    partials = sc_fn(haug, alph, edge_index, edge_attr)

    out = pl.pallas_call(
        _tc_combine,
        grid=(10,),
        in_specs=[
            pl.BlockSpec((1000, AUG), lambda i: (i, 0)),
            pl.BlockSpec((1000, AUG), lambda i: (i, 0)),
            pl.BlockSpec((1000, AUG), lambda i: (i, 0)),
            pl.BlockSpec((8, 128), lambda i: (0, 0)),
        ],
        out_specs=pl.BlockSpec((1000, 128), lambda i: (i, 0)),
        out_shape=jax.ShapeDtypeStruct((N, 128), jnp.float32),
    )(partials[0], partials[1], haug, expand)
    return out
